# baseline (speedup 1.0000x reference)
import numpy as np
import jax
import jax.numpy as jnp
from jax.experimental import pallas as pl
from jax.experimental.pallas import tpu as pltpu

_M, _P, _W, _R = 512, 256, 3, 128
_B = 32
_NPAIR = _W * _M // (2 * _R)


def _dft_consts():
    kP = np.arange(_P)
    FP = np.exp(-2j * np.pi * np.outer(kP, kP) / _P)
    kR = np.arange(_R)
    FR = np.exp(-2j * np.pi * np.outer(kR, kR) / _R)
    G = np.exp(+2j * np.pi * np.outer(kP, kP) / _P) / _P
    f32 = np.float32
    return (FP.real.astype(f32), FP.imag.astype(f32),
            FR.real.astype(f32), FR.imag.astype(f32),
            G.real.astype(f32), G.imag.astype(f32))


_FPR, _FPI, _FRR, _FRI, _GR, _GI = _dft_consts()


def _body(qt_ref, hn_r_ref, hn_i_ref, fpr_ref, fpi_ref, frcat1_ref,
          frcat2_ref, gr_ref, gi_ref, gs_ref, o_ref, hs_r, hs_i):
    f32 = jnp.float32
    bf = jnp.bfloat16

    @pl.when(pl.program_id(0) == 0)
    def _init_h():
        for p in range(_NPAIR):
            cs = slice(2 * _R * p, 2 * _R * (p + 1))
            hs_r[:, cs] = hn_r_ref[cs, :].T.astype(bf)
            hs_i[:, cs] = hn_i_ref[cs, :].T.astype(bf)

    qt = qt_ref[...].T
    tr = jnp.dot(fpr_ref[...], qt, preferred_element_type=f32)
    ti = jnp.dot(fpi_ref[...], qt, preferred_element_type=f32)
    u1 = jnp.dot(tr, frcat1_ref[...], preferred_element_type=f32)
    u2 = jnp.dot(ti, frcat2_ref[...], preferred_element_type=f32)
    qfr = u1[:, :_R] - u2[:, :_R]
    qfi = u1[:, _R:] + u2[:, _R:]
    qfr_b = qfr.astype(bf)
    qfi_b = qfi.astype(bf)
    q2r = jnp.concatenate([qfr_b, qfr_b], axis=1)
    q2i = jnp.concatenate([qfi_b, qfi_b], axis=1)
    gr = gr_ref[...]
    gi = gi_ref[...]
    gs = gs_ref[...]
    parts = []
    for p in range(_NPAIR):
        hr = hs_r[:, 2 * _R * p:2 * _R * (p + 1)]
        hi = hs_i[:, 2 * _R * p:2 * _R * (p + 1)]
        zr = q2r * hr - q2i * hi
        zi = q2r * hi + q2i * hr
        m1 = jnp.dot(gr, zr, preferred_element_type=f32)
        m2 = jnp.dot(gi, zi, preferred_element_type=f32)
        m3 = jnp.dot(gs, zr + zi, preferred_element_type=f32)
        rr = m1 - m2
        ri = m3 - m1 - m2
        mag2 = rr * rr + ri * ri + f32(1e-37)
        mag = mag2 * jax.lax.rsqrt(mag2)
        parts.append(jnp.sum(mag, axis=0))
    tot = jnp.concatenate(parts)
    tot = (tot[0:_M] + tot[_M:2 * _M] + tot[2 * _M:3 * _M]) * f32(1.0 / (_P * _W))
    o_ref[0, 0, :] = jnp.where(tot > f32(0.3), tot, f32(0.0))


def kernel(stimulus, H_real, H_imag):
    q = stimulus.reshape(_B * _R, _P)
    hn_r = jnp.transpose(H_real, (2, 0, 1)).reshape(_W * _M, _P)
    hn_i = jnp.transpose(H_imag, (2, 0, 1)).reshape(_W * _M, _P)
    const_spec = lambda shape: pl.BlockSpec(shape, lambda b: (0,) * len(shape))
    out = pl.pallas_call(
        _body,
        grid=(_B,),
        in_specs=[
            pl.BlockSpec((_R, _P), lambda b: (b, 0)),
            const_spec((_W * _M, _P)),
            const_spec((_W * _M, _P)),
            const_spec((_P, _P)),
            const_spec((_P, _P)),
            const_spec((_R, _P)),
            const_spec((_R, _P)),
            const_spec((_P, _P)),
            const_spec((_P, _P)),
            const_spec((_P, _P)),
        ],
        out_specs=pl.BlockSpec((1, 1, _M), lambda b: (b, 0, 0)),
        out_shape=jax.ShapeDtypeStruct((_B, 1, _M), jnp.float32),
        scratch_shapes=[
            pltpu.VMEM((_P, _W * _M), jnp.bfloat16),
            pltpu.VMEM((_P, _W * _M), jnp.bfloat16),
        ],
        compiler_params=pltpu.CompilerParams(
            dimension_semantics=("arbitrary",),
        ),
        name="holographic_retrieve",
    )(q, hn_r, hn_i,
      jnp.asarray(_FPR), jnp.asarray(_FPI),
      jnp.asarray(np.concatenate([_FRR, _FRI], axis=1)),
      jnp.asarray(np.concatenate([_FRI, _FRR], axis=1)),
      jnp.asarray(_GR).astype(jnp.bfloat16), jnp.asarray(_GI).astype(jnp.bfloat16),
      jnp.asarray(_GR + _GI).astype(jnp.bfloat16))
    return out.reshape(_B, _M)


# speedup vs baseline: 1.0136x; 1.0136x over previous
import numpy as np
import jax
import jax.numpy as jnp
from jax.experimental import pallas as pl
from jax.experimental.pallas import tpu as pltpu

_M, _P, _W, _R = 512, 256, 3, 128
_B = 32
_NPAIR = _W * _M // (2 * _R)


def _dft_consts():
    kP = np.arange(_P)
    FP = np.exp(-2j * np.pi * np.outer(kP, kP) / _P)
    kR = np.arange(_R)
    FR = np.exp(-2j * np.pi * np.outer(kR, kR) / _R)
    G = np.exp(+2j * np.pi * np.outer(kP, kP) / _P) / _P
    f32 = np.float32
    return (FP.real.astype(f32), FP.imag.astype(f32),
            FR.real.astype(f32), FR.imag.astype(f32),
            G.real.astype(f32), G.imag.astype(f32))


_FPR, _FPI, _FRR, _FRI, _GR, _GI = _dft_consts()


def _body(qt_ref, hn_r_ref, hn_i_ref, fpr_ref, fpi_ref, frcat1_ref,
          frcat2_ref, gr_ref, gi_ref, gs_ref, o_ref, hs_r, hs_i):
    f32 = jnp.float32
    bf = jnp.bfloat16

    @pl.when(pl.program_id(0) == 0)
    def _init_h():
        for p in range(_NPAIR):
            cs = slice(2 * _R * p, 2 * _R * (p + 1))
            hs_r[:, cs] = hn_r_ref[cs, :].T.astype(bf)
            hs_i[:, cs] = hn_i_ref[cs, :].T.astype(bf)

    qt = qt_ref[0].T
    tr = jnp.dot(fpr_ref[...], qt, preferred_element_type=f32)
    ti = jnp.dot(fpi_ref[...], qt, preferred_element_type=f32)
    u1 = jnp.dot(tr, frcat1_ref[...], preferred_element_type=f32)
    u2 = jnp.dot(ti, frcat2_ref[...], preferred_element_type=f32)
    qfr = u1[:, :_R] - u2[:, :_R]
    qfi = u1[:, _R:] + u2[:, _R:]
    qfr_b = qfr.astype(bf)
    qfi_b = qfi.astype(bf)
    q2r = jnp.concatenate([qfr_b, qfr_b], axis=1)
    q2i = jnp.concatenate([qfi_b, qfi_b], axis=1)
    gr = gr_ref[...]
    gi = gi_ref[...]
    gs = gs_ref[...]
    parts = []
    for p in range(_NPAIR):
        hr = hs_r[:, 2 * _R * p:2 * _R * (p + 1)]
        hi = hs_i[:, 2 * _R * p:2 * _R * (p + 1)]
        zr = q2r * hr - q2i * hi
        zi = q2r * hi + q2i * hr
        m1 = jnp.dot(gr, zr, preferred_element_type=f32)
        m2 = jnp.dot(gi, zi, preferred_element_type=f32)
        m3 = jnp.dot(gs, zr + zi, preferred_element_type=f32)
        rr = m1 - m2
        ri = m3 - m1 - m2
        mag2 = rr * rr + ri * ri + f32(1e-37)
        mag = mag2 * jax.lax.rsqrt(mag2)
        parts.append(jnp.sum(mag, axis=0))
    tot = jnp.concatenate(parts)
    tot = (tot[0:_M] + tot[_M:2 * _M] + tot[2 * _M:3 * _M]) * f32(1.0 / (_P * _W))
    o_ref[0, 0, :] = jnp.where(tot > f32(0.3), tot, f32(0.0))


def kernel(stimulus, H_real, H_imag):
    q = stimulus.reshape(_B, _R, _P)
    hn_r = jnp.transpose(H_real, (2, 0, 1)).reshape(_W * _M, _P)
    hn_i = jnp.transpose(H_imag, (2, 0, 1)).reshape(_W * _M, _P)
    const_spec = lambda shape: pl.BlockSpec(shape, lambda b: (0,) * len(shape))
    out = pl.pallas_call(
        _body,
        grid=(_B,),
        in_specs=[
            pl.BlockSpec((1, _R, _P), lambda b: (b, 0, 0)),
            const_spec((_W * _M, _P)),
            const_spec((_W * _M, _P)),
            const_spec((_P, _P)),
            const_spec((_P, _P)),
            const_spec((_R, _P)),
            const_spec((_R, _P)),
            const_spec((_P, _P)),
            const_spec((_P, _P)),
            const_spec((_P, _P)),
        ],
        out_specs=pl.BlockSpec((1, 1, _M), lambda b: (b, 0, 0)),
        out_shape=jax.ShapeDtypeStruct((_B, 1, _M), jnp.float32),
        scratch_shapes=[
            pltpu.VMEM((_P, _W * _M), jnp.bfloat16),
            pltpu.VMEM((_P, _W * _M), jnp.bfloat16),
        ],
        compiler_params=pltpu.CompilerParams(
            dimension_semantics=("arbitrary",),
        ),
        name="holographic_retrieve",
    )(q, hn_r, hn_i,
      jnp.asarray(_FPR), jnp.asarray(_FPI),
      jnp.asarray(np.concatenate([_FRR, _FRI], axis=1)),
      jnp.asarray(np.concatenate([_FRI, _FRR], axis=1)),
      jnp.asarray(_GR).astype(jnp.bfloat16), jnp.asarray(_GI).astype(jnp.bfloat16),
      jnp.asarray(_GR + _GI).astype(jnp.bfloat16))
    return out.reshape(_B, _M)
